# Initial kernel scaffold; baseline (speedup 1.0000x reference)
import jax
import jax.numpy as jnp
from jax.experimental import pallas as pl


def kernel(x_batch, edge_index0, edge_index1, size0, size1, W1, b1, g1, bt1, W2, b2, W3, b3, g2, bt2, W4, b4):
    raise NotImplementedError("write your pallas kernel here")


# speedup vs baseline: 2.6136x; 2.6136x over previous
import functools

import jax
import jax.numpy as jnp
from jax import lax
from jax.experimental import pallas as pl
from jax.experimental.pallas import tpu as pltpu
from jax.experimental.pallas import tpu_sc as plsc

_N0 = 270336
_N1 = 24576
_N2 = 4096
_E0 = 245760
_E1 = 20480
_D = 128
_H = 128
_C = 41
_NC = 2
_NS = 16
_LN = 16
_HALF = _D // 2
_CH = 128


def _sc_aggregate(table2, src, dst, n_out, n_edges):
    e_per_sub = n_edges // _NS
    n_blocks = e_per_sub // _CH
    rps = n_out // _NS
    mesh = plsc.VectorSubcoreMesh(core_axis_name="c", subcore_axis_name="s")

    @functools.partial(
        pl.kernel,
        out_type=jax.ShapeDtypeStruct((_NC, n_out, _HALF), jnp.float32),
        mesh=mesh,
        scratch_types=[
            pltpu.VMEM((_CH,), jnp.int32),
            pltpu.VMEM((1, _CH), jnp.int32),
            pltpu.VMEM((_CH, _HALF), jnp.float32),
            pltpu.VMEM_SHARED((n_out, _HALF), jnp.float32),
            pltpu.SemaphoreType.DMA,
        ],
        compiler_params=pltpu.CompilerParams(use_tc_tiling_on_sc=False),
    )
    def agg(table_hbm, src_hbm, dst_hbm, out_hbm, src_v, dst_v, rows_v, acc_sh, sem):
        c = lax.axis_index("c")
        s = lax.axis_index("s")

        @pl.loop(0, _CH)
        def _(i):
            @pl.loop(0, _HALF, step=_LN)
            def _(j):
                rows_v[i, pl.ds(j, _LN)] = jnp.zeros((_LN,), jnp.float32)

        @pl.loop(0, rps, step=_CH)
        def _(t):
            pltpu.sync_copy(rows_v, acc_sh.at[pl.ds(s * rps + t, _CH)])

        plsc.subcore_barrier()

        base_e = s * e_per_sub

        @pl.loop(0, n_blocks)
        def _(b):
            off = base_e + b * _CH
            pltpu.sync_copy(src_hbm.at[pl.ds(off, _CH)], src_v)
            pltpu.sync_copy(dst_hbm.at[pl.ds(off, _CH)], dst_v.at[0])

            @pl.loop(0, _CH, step=_LN)
            def _(j):
                src_v[pl.ds(j, _LN)] = src_v[pl.ds(j, _LN)] * 2 + c

            pltpu.async_copy(table_hbm.at[src_v], rows_v, sem).wait()
            pltpu.sync_copy(rows_v, acc_sh.at[dst_v.at[0]], add=True)

        plsc.subcore_barrier()

        pltpu.sync_copy(acc_sh.at[pl.ds(s * rps, rps)],
                        out_hbm.at[c, pl.ds(s * rps, rps)])

    return agg(table2, src, dst)


_BA = 2048


def _mlp0(x_batch, aggr, W1, b1, g1, bt1, W2, b2):
    nb = _N1 // _BA

    def body_a(x_ref, a_ref, w1_ref, b1_ref, h_ref, sum_ref, sq_ref):
        i = pl.program_id(0)
        a = a_ref[...]
        z = x_ref[...] + jnp.concatenate([a[0], a[1]], axis=-1)
        h = jnp.dot(z, w1_ref[...], preferred_element_type=jnp.float32) + b1_ref[...]
        h_ref[...] = h

        @pl.when(i == 0)
        def _():
            sum_ref[...] = jnp.zeros_like(sum_ref)
            sq_ref[...] = jnp.zeros_like(sq_ref)

        sum_ref[...] += jnp.sum(h, axis=0, keepdims=True)
        sq_ref[...] += jnp.sum(h * h, axis=0, keepdims=True)

    h1, hsum, hsq = pl.pallas_call(
        body_a,
        grid=(nb,),
        in_specs=[
            pl.BlockSpec((_BA, _D), lambda i: (i, 0)),
            pl.BlockSpec((_NC, _BA, _HALF), lambda i: (0, i, 0)),
            pl.BlockSpec((_D, _H), lambda i: (0, 0)),
            pl.BlockSpec((1, _H), lambda i: (0, 0)),
        ],
        out_specs=[
            pl.BlockSpec((_BA, _H), lambda i: (i, 0)),
            pl.BlockSpec((1, _H), lambda i: (0, 0)),
            pl.BlockSpec((1, _H), lambda i: (0, 0)),
        ],
        out_shape=[
            jax.ShapeDtypeStruct((_N1, _H), jnp.float32),
            jax.ShapeDtypeStruct((1, _H), jnp.float32),
            jax.ShapeDtypeStruct((1, _H), jnp.float32),
        ],
    )(x_batch, aggr, W1, b1)

    def body_b(h_ref, sum_ref, sq_ref, g_ref, bt_ref, w2_ref, b2_ref, o_ref):
        n = jnp.float32(_N1)
        m = sum_ref[...] / n
        v = sq_ref[...] / n - m * m
        h = (h_ref[...] - m) * (g_ref[...] / jnp.sqrt(v + 1e-5)) + bt_ref[...]
        h = jnp.maximum(h, 0.0)
        h = jnp.dot(h, w2_ref[...], preferred_element_type=jnp.float32) + b2_ref[...]
        o_ref[...] = jnp.maximum(h, 0.0)

    return pl.pallas_call(
        body_b,
        grid=(nb,),
        in_specs=[
            pl.BlockSpec((_BA, _H), lambda i: (i, 0)),
            pl.BlockSpec((1, _H), lambda i: (0, 0)),
            pl.BlockSpec((1, _H), lambda i: (0, 0)),
            pl.BlockSpec((1, _H), lambda i: (0, 0)),
            pl.BlockSpec((1, _H), lambda i: (0, 0)),
            pl.BlockSpec((_H, _H), lambda i: (0, 0)),
            pl.BlockSpec((1, _H), lambda i: (0, 0)),
        ],
        out_specs=pl.BlockSpec((_BA, _H), lambda i: (i, 0)),
        out_shape=jax.ShapeDtypeStruct((_N1, _H), jnp.float32),
    )(h1, hsum, hsq, g1, bt1, W2, b2)


def _mlp1(h, aggr, W3, b3, g2, bt2, W4, b4):
    def body(h_ref, a_ref, w3_ref, b3_ref, g_ref, bt_ref, w4_ref, b4_ref, o_ref):
        a = a_ref[...]
        z = h_ref[...] + jnp.concatenate([a[0], a[1]], axis=-1)
        t = jnp.dot(z, w3_ref[...], preferred_element_type=jnp.float32) + b3_ref[...]
        m = jnp.mean(t, axis=0, keepdims=True)
        v = jnp.mean((t - m) ** 2, axis=0, keepdims=True)
        t = (t - m) / jnp.sqrt(v + 1e-5) * g_ref[...] + bt_ref[...]
        t = jnp.maximum(t, 0.0)
        t = jnp.dot(t, w4_ref[...], preferred_element_type=jnp.float32) + b4_ref[...]
        t = t - jnp.max(t, axis=-1, keepdims=True)
        o_ref[...] = t - jnp.log(jnp.sum(jnp.exp(t), axis=-1, keepdims=True))

    return pl.pallas_call(
        body,
        grid=(1,),
        in_specs=[
            pl.BlockSpec((_N2, _H), lambda i: (0, 0)),
            pl.BlockSpec((_NC, _N2, _HALF), lambda i: (0, 0, 0)),
            pl.BlockSpec((_H, _H), lambda i: (0, 0)),
            pl.BlockSpec((1, _H), lambda i: (0, 0)),
            pl.BlockSpec((1, _H), lambda i: (0, 0)),
            pl.BlockSpec((1, _H), lambda i: (0, 0)),
            pl.BlockSpec((_H, _C), lambda i: (0, 0)),
            pl.BlockSpec((1, _C), lambda i: (0, 0)),
        ],
        out_specs=pl.BlockSpec((_N2, _C), lambda i: (0, 0)),
        out_shape=jax.ShapeDtypeStruct((_N2, _C), jnp.float32),
    )(h, aggr, W3, b3, g2, bt2, W4, b4)


def kernel(x_batch, edge_index0, edge_index1, size0, size1,
           W1, b1, g1, bt1, W2, b2, W3, b3, g2, bt2, W4, b4):
    x2 = x_batch.reshape(2 * _N0, _HALF)
    aggr0 = _sc_aggregate(x2, edge_index0[0], edge_index0[1], _N1, _E0)
    h = _mlp0(x_batch, aggr0, W1, b1.reshape(1, _H), g1.reshape(1, _H),
              bt1.reshape(1, _H), W2, b2.reshape(1, _H))
    h2 = h.reshape(2 * _N1, _HALF)
    aggr1 = _sc_aggregate(h2, edge_index1[0], edge_index1[1], _N2, _E1)
    return _mlp1(h, aggr1, W3, b3.reshape(1, _H), g2.reshape(1, _H),
                 bt2.reshape(1, _H), W4, b4.reshape(1, _C))


# speedup vs baseline: 4.4714x; 1.7108x over previous
import functools

import jax
import jax.numpy as jnp
from jax import lax
from jax.experimental import pallas as pl
from jax.experimental.pallas import tpu as pltpu
from jax.experimental.pallas import tpu_sc as plsc

_N0 = 270336
_N1 = 24576
_N2 = 4096
_E0 = 245760
_E1 = 20480
_D = 128
_H = 128
_C = 41
_NC = 2
_NS = 16
_LN = 16
_HALF = _D // 2
_CH = 128


def _sc_aggregate(table2, src2, dst2, n_out, n_edges, cb):
    e_per_sub = n_edges // _NS
    n_blocks = e_per_sub // _CH
    nq = n_blocks // cb
    rps = n_out // _NS
    nz = rps // _CH
    mesh = plsc.VectorSubcoreMesh(core_axis_name="c", subcore_axis_name="s")

    @functools.partial(
        pl.kernel,
        out_type=jax.ShapeDtypeStruct((_NC, n_out, _HALF), jnp.float32),
        mesh=mesh,
        scratch_types=[
            pltpu.VMEM((2, cb * _CH), jnp.int32),
            pltpu.VMEM((2, cb, _CH), jnp.int32),
            pltpu.VMEM((2, _CH, _HALF), jnp.float32),
            pltpu.VMEM_SHARED((n_out, _HALF), jnp.float32),
            pltpu.SemaphoreType.DMA((2,)),
            pltpu.SemaphoreType.DMA((2,)),
            pltpu.SemaphoreType.DMA((2,)),
        ],
        compiler_params=pltpu.CompilerParams(use_tc_tiling_on_sc=False),
    )
    def agg(table_hbm, src_hbm, dst_hbm, out_hbm, src_v, dst_v, rows_v,
            acc_sh, isem, gsem, ssem):
        c = lax.axis_index("c")
        s = lax.axis_index("s")

        def fire_idx(q, ib):
            pltpu.async_copy(
                src_hbm.at[c, pl.ds(s * e_per_sub + q * cb * _CH, cb * _CH)],
                src_v.at[ib], isem.at[ib])
            pltpu.async_copy(
                dst_hbm.at[pl.ds(s * n_blocks + q * cb, cb)],
                dst_v.at[ib], isem.at[ib])

        def drain_idx(q, ib):
            pltpu.make_async_copy(
                src_hbm.at[c, pl.ds(s * e_per_sub + q * cb * _CH, cb * _CH)],
                src_v.at[ib], isem.at[ib]).wait()
            pltpu.make_async_copy(
                dst_hbm.at[pl.ds(s * n_blocks + q * cb, cb)],
                dst_v.at[ib], isem.at[ib]).wait()

        fire_idx(0, 0)

        @pl.loop(0, _CH)
        def _(i):
            @pl.loop(0, _HALF, step=_LN)
            def _(j):
                rows_v[0, i, pl.ds(j, _LN)] = jnp.zeros((_LN,), jnp.float32)

        for t in range(nz):
            pltpu.async_copy(
                rows_v.at[0], acc_sh.at[pl.ds((s * nz + t) * _CH, _CH)],
                ssem.at[0])
        for t in range(nz):
            pltpu.make_async_copy(
                rows_v.at[0], acc_sh.at[pl.ds(s * nz * _CH, _CH)],
                ssem.at[0]).wait()

        plsc.subcore_barrier()

        for q in range(nq):
            ib = q % 2
            if q + 1 < nq:
                fire_idx(q + 1, 1 - ib)
            drain_idx(q, ib)

            @pl.loop(0, cb // 2)
            def _(t):
                gathers = []
                for k in range(2):
                    j = 2 * t + k

                    @pl.when(t > 0)
                    def _():
                        pltpu.make_async_copy(
                            rows_v.at[k], acc_sh.at[dst_v.at[ib, j - 2]],
                            ssem.at[k]).wait()

                    gathers.append(pltpu.async_copy(
                        table_hbm.at[src_v.at[ib, pl.ds(j * _CH, _CH)]],
                        rows_v.at[k], gsem.at[k]))
                for k in range(2):
                    j = 2 * t + k
                    gathers[k].wait()
                    pltpu.async_copy(rows_v.at[k], acc_sh.at[dst_v.at[ib, j]],
                                     ssem.at[k], add=True)

            for k in range(2):
                pltpu.make_async_copy(
                    rows_v.at[k], acc_sh.at[dst_v.at[ib, cb - 2 + k]],
                    ssem.at[k]).wait()

        plsc.subcore_barrier()

        pltpu.sync_copy(acc_sh.at[pl.ds(s * rps, rps)],
                        out_hbm.at[c, pl.ds(s * rps, rps)])

    return agg(table2, src2, dst2)


_BA = 2048


def _mlp0(x_batch, aggr, W1, b1, g1, bt1, W2, b2):
    nb = _N1 // _BA

    def body_a(x_ref, a_ref, w1_ref, b1_ref, h_ref, sum_ref, sq_ref):
        i = pl.program_id(0)
        a = a_ref[...]
        z = x_ref[...] + jnp.concatenate([a[0], a[1]], axis=-1)
        h = jnp.dot(z, w1_ref[...], preferred_element_type=jnp.float32) + b1_ref[...]
        h_ref[...] = h

        @pl.when(i == 0)
        def _():
            sum_ref[...] = jnp.zeros_like(sum_ref)
            sq_ref[...] = jnp.zeros_like(sq_ref)

        sum_ref[...] += jnp.sum(h, axis=0, keepdims=True)
        sq_ref[...] += jnp.sum(h * h, axis=0, keepdims=True)

    h1, hsum, hsq = pl.pallas_call(
        body_a,
        grid=(nb,),
        in_specs=[
            pl.BlockSpec((_BA, _D), lambda i: (i, 0)),
            pl.BlockSpec((_NC, _BA, _HALF), lambda i: (0, i, 0)),
            pl.BlockSpec((_D, _H), lambda i: (0, 0)),
            pl.BlockSpec((1, _H), lambda i: (0, 0)),
        ],
        out_specs=[
            pl.BlockSpec((_BA, _H), lambda i: (i, 0)),
            pl.BlockSpec((1, _H), lambda i: (0, 0)),
            pl.BlockSpec((1, _H), lambda i: (0, 0)),
        ],
        out_shape=[
            jax.ShapeDtypeStruct((_N1, _H), jnp.float32),
            jax.ShapeDtypeStruct((1, _H), jnp.float32),
            jax.ShapeDtypeStruct((1, _H), jnp.float32),
        ],
    )(x_batch, aggr, W1, b1)

    def body_b(h_ref, sum_ref, sq_ref, g_ref, bt_ref, w2_ref, b2_ref, o_ref):
        n = jnp.float32(_N1)
        m = sum_ref[...] / n
        v = sq_ref[...] / n - m * m
        h = (h_ref[...] - m) * (g_ref[...] / jnp.sqrt(v + 1e-5)) + bt_ref[...]
        h = jnp.maximum(h, 0.0)
        h = jnp.dot(h, w2_ref[...], preferred_element_type=jnp.float32) + b2_ref[...]
        o_ref[...] = jnp.maximum(h, 0.0)

    return pl.pallas_call(
        body_b,
        grid=(nb,),
        in_specs=[
            pl.BlockSpec((_BA, _H), lambda i: (i, 0)),
            pl.BlockSpec((1, _H), lambda i: (0, 0)),
            pl.BlockSpec((1, _H), lambda i: (0, 0)),
            pl.BlockSpec((1, _H), lambda i: (0, 0)),
            pl.BlockSpec((1, _H), lambda i: (0, 0)),
            pl.BlockSpec((_H, _H), lambda i: (0, 0)),
            pl.BlockSpec((1, _H), lambda i: (0, 0)),
        ],
        out_specs=pl.BlockSpec((_BA, _H), lambda i: (i, 0)),
        out_shape=jax.ShapeDtypeStruct((_N1, _H), jnp.float32),
    )(h1, hsum, hsq, g1, bt1, W2, b2)


def _mlp1(h, aggr, W3, b3, g2, bt2, W4, b4):
    def body(h_ref, a_ref, w3_ref, b3_ref, g_ref, bt_ref, w4_ref, b4_ref, o_ref):
        a = a_ref[...]
        z = h_ref[...] + jnp.concatenate([a[0], a[1]], axis=-1)
        t = jnp.dot(z, w3_ref[...], preferred_element_type=jnp.float32) + b3_ref[...]
        m = jnp.mean(t, axis=0, keepdims=True)
        v = jnp.mean((t - m) ** 2, axis=0, keepdims=True)
        t = (t - m) / jnp.sqrt(v + 1e-5) * g_ref[...] + bt_ref[...]
        t = jnp.maximum(t, 0.0)
        t = jnp.dot(t, w4_ref[...], preferred_element_type=jnp.float32) + b4_ref[...]
        t = t - jnp.max(t, axis=-1, keepdims=True)
        o_ref[...] = t - jnp.log(jnp.sum(jnp.exp(t), axis=-1, keepdims=True))

    return pl.pallas_call(
        body,
        grid=(1,),
        in_specs=[
            pl.BlockSpec((_N2, _H), lambda i: (0, 0)),
            pl.BlockSpec((_NC, _N2, _HALF), lambda i: (0, 0, 0)),
            pl.BlockSpec((_H, _H), lambda i: (0, 0)),
            pl.BlockSpec((1, _H), lambda i: (0, 0)),
            pl.BlockSpec((1, _H), lambda i: (0, 0)),
            pl.BlockSpec((1, _H), lambda i: (0, 0)),
            pl.BlockSpec((_H, _C), lambda i: (0, 0)),
            pl.BlockSpec((1, _C), lambda i: (0, 0)),
        ],
        out_specs=pl.BlockSpec((_N2, _C), lambda i: (0, 0)),
        out_shape=jax.ShapeDtypeStruct((_N2, _C), jnp.float32),
    )(h, aggr, W3, b3, g2, bt2, W4, b4)


def kernel(x_batch, edge_index0, edge_index1, size0, size1,
           W1, b1, g1, bt1, W2, b2, W3, b3, g2, bt2, W4, b4):
    x2 = x_batch.reshape(2 * _N0, _HALF)
    src0d = 2 * edge_index0[0]
    aggr0 = _sc_aggregate(x2, jnp.stack([src0d, src0d + 1]),
                          edge_index0[1].reshape(_E0 // _CH, _CH), _N1, _E0, 12)
    h = _mlp0(x_batch, aggr0, W1, b1.reshape(1, _H), g1.reshape(1, _H),
              bt1.reshape(1, _H), W2, b2.reshape(1, _H))
    h2 = h.reshape(2 * _N1, _HALF)
    src1d = 2 * edge_index1[0]
    aggr1 = _sc_aggregate(h2, jnp.stack([src1d, src1d + 1]),
                          edge_index1[1].reshape(_E1 // _CH, _CH), _N2, _E1, 10)
    return _mlp1(h, aggr1, W3, b3.reshape(1, _H), g2.reshape(1, _H),
                 bt2.reshape(1, _H), W4, b4.reshape(1, _C))


# speedup vs baseline: 7.1512x; 1.5993x over previous
import functools

import jax
import jax.numpy as jnp
from jax import lax
from jax.experimental import pallas as pl
from jax.experimental.pallas import tpu as pltpu
from jax.experimental.pallas import tpu_sc as plsc

_N0 = 270336
_N1 = 24576
_N2 = 4096
_E0 = 245760
_E1 = 20480
_D = 128
_H = 128
_C = 41
_NC = 2
_NS = 16
_LN = 16
_HALF = _D // 2
_CH = 128


def _sc_aggregate(table2, src2, dst2, n_out, n_edges, cb, nbuf):
    e_per_sub = n_edges // _NS
    n_blocks = e_per_sub // _CH
    nq = n_blocks // cb
    rps = n_out // _NS
    nz = rps // _CH
    mesh = plsc.VectorSubcoreMesh(core_axis_name="c", subcore_axis_name="s")

    @functools.partial(
        pl.kernel,
        out_type=jax.ShapeDtypeStruct((_NC, n_out, _HALF), jnp.float32),
        mesh=mesh,
        scratch_types=[
            pltpu.VMEM((2, cb * _CH), jnp.int32),
            pltpu.VMEM((2, cb, _CH), jnp.int32),
            pltpu.VMEM((nbuf, _CH, _HALF), jnp.float32),
            pltpu.VMEM_SHARED((n_out, _HALF), jnp.float32),
            pltpu.SemaphoreType.DMA((2,)),
            pltpu.SemaphoreType.DMA((nbuf,)),
            pltpu.SemaphoreType.DMA((nbuf,)),
        ],
        compiler_params=pltpu.CompilerParams(use_tc_tiling_on_sc=False),
    )
    def agg(table_hbm, src_hbm, dst_hbm, out_hbm, src_v, dst_v, rows_v,
            acc_sh, isem, gsem, ssem):
        c = lax.axis_index("c")
        s = lax.axis_index("s")

        def fire_idx(q, ib):
            pltpu.async_copy(
                src_hbm.at[c, pl.ds(s * e_per_sub + q * cb * _CH, cb * _CH)],
                src_v.at[ib], isem.at[ib])
            pltpu.async_copy(
                dst_hbm.at[pl.ds(s * n_blocks + q * cb, cb)],
                dst_v.at[ib], isem.at[ib])

        def drain_idx(q, ib):
            pltpu.make_async_copy(
                src_hbm.at[c, pl.ds(s * e_per_sub + q * cb * _CH, cb * _CH)],
                src_v.at[ib], isem.at[ib]).wait()
            pltpu.make_async_copy(
                dst_hbm.at[pl.ds(s * n_blocks + q * cb, cb)],
                dst_v.at[ib], isem.at[ib]).wait()

        fire_idx(0, 0)

        @pl.loop(0, _CH)
        def _(i):
            @pl.loop(0, _HALF, step=_LN)
            def _(j):
                rows_v[0, i, pl.ds(j, _LN)] = jnp.zeros((_LN,), jnp.float32)

        for t in range(nz):
            pltpu.async_copy(
                rows_v.at[0], acc_sh.at[pl.ds((s * nz + t) * _CH, _CH)],
                ssem.at[0])
        for t in range(nz):
            pltpu.make_async_copy(
                rows_v.at[0], acc_sh.at[pl.ds(s * nz * _CH, _CH)],
                ssem.at[0]).wait()

        plsc.subcore_barrier()

        for q in range(nq):
            ib = q % 2
            if q + 1 < nq:
                fire_idx(q + 1, 1 - ib)
            drain_idx(q, ib)

            @pl.loop(0, cb // nbuf)
            def _(t):
                gathers = []
                for k in range(nbuf):
                    j = nbuf * t + k

                    @pl.when(t > 0)
                    def _():
                        pltpu.make_async_copy(
                            rows_v.at[k], acc_sh.at[dst_v.at[ib, j - nbuf]],
                            ssem.at[k]).wait()

                    gathers.append(pltpu.async_copy(
                        table_hbm.at[src_v.at[ib, pl.ds(j * _CH, _CH)]],
                        rows_v.at[k], gsem.at[k]))
                for k in range(nbuf):
                    j = nbuf * t + k
                    gathers[k].wait()
                    pltpu.async_copy(rows_v.at[k], acc_sh.at[dst_v.at[ib, j]],
                                     ssem.at[k], add=True)

            for k in range(nbuf):
                pltpu.make_async_copy(
                    rows_v.at[k], acc_sh.at[dst_v.at[ib, cb - nbuf + k]],
                    ssem.at[k]).wait()

        plsc.subcore_barrier()

        pltpu.sync_copy(acc_sh.at[pl.ds(s * rps, rps)],
                        out_hbm.at[c, pl.ds(s * rps, rps)])

    return agg(table2, src2, dst2)


_BA = 2048


def _mlp0(x_batch, aggr, W1, b1, g1, bt1, W2, b2):
    nb = _N1 // _BA

    def body_a(x_ref, a_ref, w1_ref, b1_ref, h_ref, sum_ref, sq_ref):
        i = pl.program_id(0)
        a = a_ref[...]
        z = x_ref[...] + jnp.concatenate([a[0], a[1]], axis=-1)
        h = jnp.dot(z, w1_ref[...], preferred_element_type=jnp.float32) + b1_ref[...]
        h_ref[...] = h

        @pl.when(i == 0)
        def _():
            sum_ref[...] = jnp.zeros_like(sum_ref)
            sq_ref[...] = jnp.zeros_like(sq_ref)

        sum_ref[...] += jnp.sum(h, axis=0, keepdims=True)
        sq_ref[...] += jnp.sum(h * h, axis=0, keepdims=True)

    h1, hsum, hsq = pl.pallas_call(
        body_a,
        grid=(nb,),
        in_specs=[
            pl.BlockSpec((_BA, _D), lambda i: (i, 0)),
            pl.BlockSpec((_NC, _BA, _HALF), lambda i: (0, i, 0)),
            pl.BlockSpec((_D, _H), lambda i: (0, 0)),
            pl.BlockSpec((1, _H), lambda i: (0, 0)),
        ],
        out_specs=[
            pl.BlockSpec((_BA, _H), lambda i: (i, 0)),
            pl.BlockSpec((1, _H), lambda i: (0, 0)),
            pl.BlockSpec((1, _H), lambda i: (0, 0)),
        ],
        out_shape=[
            jax.ShapeDtypeStruct((_N1, _H), jnp.float32),
            jax.ShapeDtypeStruct((1, _H), jnp.float32),
            jax.ShapeDtypeStruct((1, _H), jnp.float32),
        ],
    )(x_batch, aggr, W1, b1)

    def body_b(h_ref, sum_ref, sq_ref, g_ref, bt_ref, w2_ref, b2_ref, o_ref):
        n = jnp.float32(_N1)
        m = sum_ref[...] / n
        v = sq_ref[...] / n - m * m
        h = (h_ref[...] - m) * (g_ref[...] / jnp.sqrt(v + 1e-5)) + bt_ref[...]
        h = jnp.maximum(h, 0.0)
        h = jnp.dot(h, w2_ref[...], preferred_element_type=jnp.float32) + b2_ref[...]
        o_ref[...] = jnp.maximum(h, 0.0)

    return pl.pallas_call(
        body_b,
        grid=(nb,),
        in_specs=[
            pl.BlockSpec((_BA, _H), lambda i: (i, 0)),
            pl.BlockSpec((1, _H), lambda i: (0, 0)),
            pl.BlockSpec((1, _H), lambda i: (0, 0)),
            pl.BlockSpec((1, _H), lambda i: (0, 0)),
            pl.BlockSpec((1, _H), lambda i: (0, 0)),
            pl.BlockSpec((_H, _H), lambda i: (0, 0)),
            pl.BlockSpec((1, _H), lambda i: (0, 0)),
        ],
        out_specs=pl.BlockSpec((_BA, _H), lambda i: (i, 0)),
        out_shape=jax.ShapeDtypeStruct((_N1, _H), jnp.float32),
    )(h1, hsum, hsq, g1, bt1, W2, b2)


def _mlp1(h, aggr, W3, b3, g2, bt2, W4, b4):
    def body(h_ref, a_ref, w3_ref, b3_ref, g_ref, bt_ref, w4_ref, b4_ref, o_ref):
        a = a_ref[...]
        z = h_ref[...] + jnp.concatenate([a[0], a[1]], axis=-1)
        t = jnp.dot(z, w3_ref[...], preferred_element_type=jnp.float32) + b3_ref[...]
        m = jnp.mean(t, axis=0, keepdims=True)
        v = jnp.mean((t - m) ** 2, axis=0, keepdims=True)
        t = (t - m) / jnp.sqrt(v + 1e-5) * g_ref[...] + bt_ref[...]
        t = jnp.maximum(t, 0.0)
        t = jnp.dot(t, w4_ref[...], preferred_element_type=jnp.float32) + b4_ref[...]
        t = t - jnp.max(t, axis=-1, keepdims=True)
        o_ref[...] = t - jnp.log(jnp.sum(jnp.exp(t), axis=-1, keepdims=True))

    return pl.pallas_call(
        body,
        grid=(1,),
        in_specs=[
            pl.BlockSpec((_N2, _H), lambda i: (0, 0)),
            pl.BlockSpec((_NC, _N2, _HALF), lambda i: (0, 0, 0)),
            pl.BlockSpec((_H, _H), lambda i: (0, 0)),
            pl.BlockSpec((1, _H), lambda i: (0, 0)),
            pl.BlockSpec((1, _H), lambda i: (0, 0)),
            pl.BlockSpec((1, _H), lambda i: (0, 0)),
            pl.BlockSpec((_H, _C), lambda i: (0, 0)),
            pl.BlockSpec((1, _C), lambda i: (0, 0)),
        ],
        out_specs=pl.BlockSpec((_N2, _C), lambda i: (0, 0)),
        out_shape=jax.ShapeDtypeStruct((_N2, _C), jnp.float32),
    )(h, aggr, W3, b3, g2, bt2, W4, b4)


def kernel(x_batch, edge_index0, edge_index1, size0, size1,
           W1, b1, g1, bt1, W2, b2, W3, b3, g2, bt2, W4, b4):
    x2 = x_batch.reshape(2 * _N0, _HALF)
    src0d = 2 * edge_index0[0]
    aggr0 = _sc_aggregate(x2, jnp.stack([src0d, src0d + 1]),
                          edge_index0[1].reshape(_E0 // _CH, _CH), _N1, _E0, 12, 3)
    h = _mlp0(x_batch, aggr0, W1, b1.reshape(1, _H), g1.reshape(1, _H),
              bt1.reshape(1, _H), W2, b2.reshape(1, _H))
    h2 = h.reshape(2 * _N1, _HALF)
    src1d = 2 * edge_index1[0]
    aggr1 = _sc_aggregate(h2, jnp.stack([src1d, src1d + 1]),
                          edge_index1[1].reshape(_E1 // _CH, _CH), _N2, _E1, 10, 2)
    return _mlp1(h, aggr1, W3, b3.reshape(1, _H), g2.reshape(1, _H),
                 bt2.reshape(1, _H), W4, b4.reshape(1, _C))
